# baseline (speedup 1.0000x reference)
import jax
import jax.numpy as jnp
from jax.experimental import pallas as pl
from jax.experimental.pallas import tpu as pltpu

_SPLITS = 4


def _fused_kernel(idx_ref, w_ref, lam_ref, x_ref, o_ref, a_ref, x2_ref):
    i = pl.program_id(0)
    j = pl.program_id(1)

    @pl.when(jnp.logical_and(i == 0, j == 0))
    def _build():
        h = a_ref.shape[0]
        k_fan = idx_ref.shape[0]
        row = jax.lax.broadcasted_iota(jnp.int32, (h, h), 0)
        mt = jnp.zeros((h, h), dtype=jnp.float32)
        for k in range(k_fan):
            hit = (row == idx_ref[k:k + 1, :]).astype(jnp.float32)
            mt = mt + w_ref[0, k] * hit
        mt5 = mt
        for _ in range(4):
            mt5 = jnp.dot(mt, mt5, preferred_element_type=jnp.float32)
        lam = lam_ref[0, 0]
        a_ref[...] = (lam * lam * lam * lam * lam) * mt5.T

    h, w, c = x_ref.shape
    hq = h // _SPLITS

    @pl.when(j == 0)
    def _stage():
        x2_ref[...] = x_ref[...].reshape(h, w * c)

    for q in range(_SPLITS):
        @pl.when(j == q)
        def _apply(q=q):
            ob = jnp.dot(a_ref[q * hq:(q + 1) * hq, :], x2_ref[...],
                         preferred_element_type=jnp.float32)
            o_ref[...] = ob.reshape(hq, w, c)


def kernel(inputs, ind1, w1, lambda1):
    b, h, w, c = inputs.shape
    k_fan = ind1.shape[0]

    idx = ind1[..., 0].astype(jnp.int32)
    wv = w1.reshape(1, k_fan).astype(jnp.float32)
    lam = lambda1.reshape(1, 1).astype(jnp.float32)

    hq = h // _SPLITS
    x3 = inputs.reshape(b * h, w, c)
    out3 = pl.pallas_call(
        _fused_kernel,
        grid=(b, _SPLITS),
        in_specs=[
            pl.BlockSpec(memory_space=pltpu.VMEM),
            pl.BlockSpec(memory_space=pltpu.SMEM),
            pl.BlockSpec(memory_space=pltpu.SMEM),
            pl.BlockSpec((h, w, c), lambda i, j: (i, 0, 0)),
        ],
        out_specs=pl.BlockSpec((hq, w, c), lambda i, j: (i * _SPLITS + j, 0, 0)),
        out_shape=jax.ShapeDtypeStruct((b * h, w, c), jnp.float32),
        scratch_shapes=[
            pltpu.VMEM((h, h), jnp.float32),
            pltpu.VMEM((h, w * c), jnp.float32),
        ],
    )(idx, wv, lam, x3)

    return out3.reshape(b, h, w, c)


# speedup vs baseline: 1.5165x; 1.5165x over previous
import jax
import jax.numpy as jnp
from jax.experimental import pallas as pl
from jax.experimental.pallas import tpu as pltpu


def _fused_kernel(idx_ref, w_ref, lam_ref, x_ref, o_ref, at_ref):
    i = pl.program_id(0)
    j = pl.program_id(1)

    @pl.when(jnp.logical_and(i == 0, j == 0))
    def _build():
        h = at_ref.shape[0]
        k_fan = idx_ref.shape[0]
        row = jax.lax.broadcasted_iota(jnp.int32, (h, h), 0)
        mt = jnp.zeros((h, h), dtype=jnp.float32)
        for k in range(k_fan):
            hit = (row == idx_ref[k:k + 1, :]).astype(jnp.float32)
            mt = mt + w_ref[0, k] * hit
        mt5 = mt
        for _ in range(4):
            mt5 = jnp.dot(mt, mt5, preferred_element_type=jnp.float32)
        lam = lam_ref[0, 0]
        at_ref[...] = (lam * lam * lam * lam * lam) * mt5

    h, wblk, c = x_ref.shape
    x2 = x_ref[...].reshape(h, wblk * c)
    ob = jax.lax.dot_general(
        at_ref[...], x2, (((0,), (0,)), ((), ())),
        preferred_element_type=jnp.float32)
    o_ref[...] = ob.reshape(h, wblk, c)


def kernel(inputs, ind1, w1, lambda1):
    b, h, w, c = inputs.shape
    k_fan = ind1.shape[0]

    idx = ind1[..., 0].astype(jnp.int32)
    wv = w1.reshape(1, k_fan).astype(jnp.float32)
    lam = lambda1.reshape(1, 1).astype(jnp.float32)

    wblk = 256
    x3 = inputs.reshape(b * h, w, c)
    out3 = pl.pallas_call(
        _fused_kernel,
        grid=(b, w // wblk),
        in_specs=[
            pl.BlockSpec(memory_space=pltpu.VMEM),
            pl.BlockSpec(memory_space=pltpu.SMEM),
            pl.BlockSpec(memory_space=pltpu.SMEM),
            pl.BlockSpec((h, wblk, c), lambda i, j: (i, j, 0)),
        ],
        out_specs=pl.BlockSpec((h, wblk, c), lambda i, j: (i, j, 0)),
        out_shape=jax.ShapeDtypeStruct((b * h, w, c), jnp.float32),
        scratch_shapes=[pltpu.VMEM((h, h), jnp.float32)],
    )(idx, wv, lam, x3)

    return out3.reshape(b, h, w, c)


# speedup vs baseline: 1.6500x; 1.0880x over previous
import jax
import jax.numpy as jnp
from jax.experimental import pallas as pl
from jax.experimental.pallas import tpu as pltpu


def _manual_kernel(idx_ref, w_ref, lam_ref, x_hbm, o_hbm,
                   in_buf, out_buf, at_ref, in_sem, out_sem):
    b, h = in_buf.shape[0], in_buf.shape[1]
    wc = in_buf.shape[2] * in_buf.shape[3]

    in_copies = []
    for i in range(b):
        cp = pltpu.make_async_copy(
            x_hbm.at[pl.ds(i * h, h)], in_buf.at[i], in_sem.at[i])
        cp.start()
        in_copies.append(cp)

    k_fan = idx_ref.shape[0]
    row = jax.lax.broadcasted_iota(jnp.int32, (h, h), 0)
    mt = jnp.zeros((h, h), dtype=jnp.float32)
    for k in range(k_fan):
        hit = (row == idx_ref[k:k + 1, :]).astype(jnp.float32)
        mt = mt + w_ref[0, k] * hit
    mt5 = mt
    for _ in range(4):
        mt5 = jnp.dot(mt, mt5, preferred_element_type=jnp.float32)
    lam = lam_ref[0, 0]
    at_ref[...] = (lam * lam * lam * lam * lam) * mt5

    out_copies = []
    for i in range(b):
        in_copies[i].wait()
        x2 = in_buf[i].reshape(h, wc)
        ob = jax.lax.dot_general(
            at_ref[...], x2, (((0,), (0,)), ((), ())),
            preferred_element_type=jnp.float32)
        out_buf[i] = ob.reshape(*in_buf.shape[1:])
        cp = pltpu.make_async_copy(
            out_buf.at[i], o_hbm.at[pl.ds(i * h, h)], out_sem.at[i])
        cp.start()
        out_copies.append(cp)
    for cp in out_copies:
        cp.wait()


def kernel(inputs, ind1, w1, lambda1):
    b, h, w, c = inputs.shape
    k_fan = ind1.shape[0]

    idx = ind1[..., 0].astype(jnp.int32)
    wv = w1.reshape(1, k_fan).astype(jnp.float32)
    lam = lambda1.reshape(1, 1).astype(jnp.float32)

    x3 = inputs.reshape(b * h, w, c)
    out3 = pl.pallas_call(
        _manual_kernel,
        in_specs=[
            pl.BlockSpec(memory_space=pltpu.VMEM),
            pl.BlockSpec(memory_space=pltpu.SMEM),
            pl.BlockSpec(memory_space=pltpu.SMEM),
            pl.BlockSpec(memory_space=pl.ANY),
        ],
        out_specs=pl.BlockSpec(memory_space=pl.ANY),
        out_shape=jax.ShapeDtypeStruct((b * h, w, c), jnp.float32),
        scratch_shapes=[
            pltpu.VMEM((b, h, w, c), jnp.float32),
            pltpu.VMEM((b, h, w, c), jnp.float32),
            pltpu.VMEM((h, h), jnp.float32),
            pltpu.SemaphoreType.DMA((b,)),
            pltpu.SemaphoreType.DMA((b,)),
        ],
    )(idx, wv, lam, x3)

    return out3.reshape(b, h, w, c)


# speedup vs baseline: 1.7217x; 1.0434x over previous
import jax
import jax.numpy as jnp
from jax.experimental import pallas as pl
from jax.experimental.pallas import tpu as pltpu


def _manual_kernel(idx_ref, w_ref, lam_ref, x_hbm, o_hbm,
                   in_buf, out_buf, at_ref, in_sem, out_sem):
    b, h = in_buf.shape[0], in_buf.shape[1]

    in_copies = []
    for i in range(b):
        cp = pltpu.make_async_copy(
            x_hbm.at[pl.ds(i * h, h)], in_buf.at[i], in_sem.at[i])
        cp.start()
        in_copies.append(cp)

    k_fan = idx_ref.shape[0]
    row = jax.lax.broadcasted_iota(jnp.int32, (h, h), 0)
    mt = jnp.zeros((h, h), dtype=jnp.float32)
    for k in range(k_fan):
        hit = (row == idx_ref[k:k + 1, :]).astype(jnp.float32)
        mt = mt + w_ref[0, k] * hit
    mt5 = mt
    for _ in range(4):
        mt5 = jnp.dot(mt, mt5, preferred_element_type=jnp.float32)
    lam = lam_ref[0, 0]
    at_ref[...] = (lam * lam * lam * lam * lam) * mt5

    wfull, c = in_buf.shape[2], in_buf.shape[3]
    whalf = wfull // 2
    out_copies = []
    for i in range(b):
        in_copies[i].wait()
        for q in range(2):
            xq = in_buf[i, :, q * whalf:(q + 1) * whalf, :]
            x2 = xq.reshape(h, whalf * c)
            ob = jax.lax.dot_general(
                at_ref[...], x2, (((0,), (0,)), ((), ())),
                preferred_element_type=jnp.float32)
            out_buf[i, :, q * whalf:(q + 1) * whalf, :] = ob.reshape(h, whalf, c)
            cp = pltpu.make_async_copy(
                out_buf.at[i, :, pl.ds(q * whalf, whalf)],
                o_hbm.at[pl.ds(i * h, h), pl.ds(q * whalf, whalf)],
                out_sem.at[i, q])
            cp.start()
            out_copies.append(cp)
    for cp in out_copies:
        cp.wait()


def kernel(inputs, ind1, w1, lambda1):
    b, h, w, c = inputs.shape
    k_fan = ind1.shape[0]

    idx = ind1[..., 0].astype(jnp.int32)
    wv = w1.reshape(1, k_fan).astype(jnp.float32)
    lam = lambda1.reshape(1, 1).astype(jnp.float32)

    x3 = inputs.reshape(b * h, w, c)
    out3 = pl.pallas_call(
        _manual_kernel,
        in_specs=[
            pl.BlockSpec(memory_space=pltpu.VMEM),
            pl.BlockSpec(memory_space=pltpu.SMEM),
            pl.BlockSpec(memory_space=pltpu.SMEM),
            pl.BlockSpec(memory_space=pl.ANY),
        ],
        out_specs=pl.BlockSpec(memory_space=pl.ANY),
        out_shape=jax.ShapeDtypeStruct((b * h, w, c), jnp.float32),
        scratch_shapes=[
            pltpu.VMEM((b, h, w, c), jnp.float32),
            pltpu.VMEM((b, h, w, c), jnp.float32),
            pltpu.VMEM((h, h), jnp.float32),
            pltpu.SemaphoreType.DMA((b,)),
            pltpu.SemaphoreType.DMA((b, 2)),
        ],
    )(idx, wv, lam, x3)

    return out3.reshape(b, h, w, c)


# speedup vs baseline: 1.7511x; 1.0171x over previous
import jax
import jax.numpy as jnp
from jax.experimental import pallas as pl
from jax.experimental.pallas import tpu as pltpu


def _manual_kernel(idx_ref, w_ref, lam_ref, x_hbm, o_hbm,
                   in_buf, out_buf, at_ref, in_sem, out_sem):
    b, h = in_buf.shape[0], in_buf.shape[1]

    in_copies = []
    for i in range(b):
        cp = pltpu.make_async_copy(
            x_hbm.at[pl.ds(i * h, h)], in_buf.at[i], in_sem.at[i])
        cp.start()
        in_copies.append(cp)

    k_fan = idx_ref.shape[0]
    row = jax.lax.broadcasted_iota(jnp.int32, (h, h), 0)
    mt = jnp.zeros((h, h), dtype=jnp.float32)
    for k in range(k_fan):
        hit = (row == idx_ref[k:k + 1, :]).astype(jnp.float32)
        mt = mt + w_ref[0, k] * hit
    mt5 = mt
    for _ in range(4):
        mt5 = jnp.dot(mt, mt5, preferred_element_type=jnp.float32)
    lam = lam_ref[0, 0]
    at_ref[...] = (lam * lam * lam * lam * lam) * mt5

    wfull, c = in_buf.shape[2], in_buf.shape[3]
    whalf = wfull // 4
    out_copies = []
    for i in range(b):
        in_copies[i].wait()
        for q in range(4):
            xq = in_buf[i, :, q * whalf:(q + 1) * whalf, :]
            x2 = xq.reshape(h, whalf * c)
            ob = jax.lax.dot_general(
                at_ref[...], x2, (((0,), (0,)), ((), ())),
                preferred_element_type=jnp.float32)
            out_buf[i, :, q * whalf:(q + 1) * whalf, :] = ob.reshape(h, whalf, c)
            cp = pltpu.make_async_copy(
                out_buf.at[i, :, pl.ds(q * whalf, whalf)],
                o_hbm.at[pl.ds(i * h, h), pl.ds(q * whalf, whalf)],
                out_sem.at[i, q])
            cp.start()
            out_copies.append(cp)
    for cp in out_copies:
        cp.wait()


def kernel(inputs, ind1, w1, lambda1):
    b, h, w, c = inputs.shape
    k_fan = ind1.shape[0]

    idx = ind1[..., 0].astype(jnp.int32)
    wv = w1.reshape(1, k_fan).astype(jnp.float32)
    lam = lambda1.reshape(1, 1).astype(jnp.float32)

    x3 = inputs.reshape(b * h, w, c)
    out3 = pl.pallas_call(
        _manual_kernel,
        in_specs=[
            pl.BlockSpec(memory_space=pltpu.VMEM),
            pl.BlockSpec(memory_space=pltpu.SMEM),
            pl.BlockSpec(memory_space=pltpu.SMEM),
            pl.BlockSpec(memory_space=pl.ANY),
        ],
        out_specs=pl.BlockSpec(memory_space=pl.ANY),
        out_shape=jax.ShapeDtypeStruct((b * h, w, c), jnp.float32),
        scratch_shapes=[
            pltpu.VMEM((b, h, w, c), jnp.float32),
            pltpu.VMEM((b, h, w, c), jnp.float32),
            pltpu.VMEM((h, h), jnp.float32),
            pltpu.SemaphoreType.DMA((b,)),
            pltpu.SemaphoreType.DMA((b, 4)),
        ],
    )(idx, wv, lam, x3)

    return out3.reshape(b, h, w, c)
